# Initial kernel scaffold; baseline (speedup 1.0000x reference)
import jax
import jax.numpy as jnp
from jax.experimental import pallas as pl


def kernel(x, Wg, W1, W2):
    raise NotImplementedError("write your pallas kernel here")


# speedup vs baseline: 3.3138x; 3.3138x over previous
import functools

import jax
import jax.numpy as jnp
from jax.experimental import pallas as pl
from jax.experimental.pallas import tpu as pltpu

TOPK = 2
BT = 256
BM = 128


def _gate_kernel(h_ref, wg_ref, e1_ref, e2_ref, w1_ref, w2_ref,
                 loadw_ref, cnt_ref, aux_ref):
    g = pl.program_id(0)
    ng = pl.num_programs(0)
    E = wg_ref.shape[1]
    logits = jnp.dot(h_ref[...], wg_ref[...],
                     preferred_element_type=jnp.float32)
    eidx = jax.lax.broadcasted_iota(jnp.int32, logits.shape, 1)
    m1 = jnp.max(logits, axis=1, keepdims=True)
    a1 = jnp.min(jnp.where(logits == m1, eidx, E), axis=1, keepdims=True)
    masked = jnp.where(eidx == a1, -jnp.inf, logits)
    m2 = jnp.max(masked, axis=1, keepdims=True)
    a2 = jnp.min(jnp.where(masked == m2, eidx, E), axis=1, keepdims=True)
    p1 = 1.0 / (1.0 + jnp.exp(m2 - m1))
    p2 = 1.0 - p1
    e1_ref[...] = a1
    e2_ref[...] = a2
    w1_ref[...] = p1
    w2_ref[...] = p2
    one1 = eidx == a1
    one2 = eidx == a2
    loadc = jnp.sum(jnp.where(one1, p1, 0.0) + jnp.where(one2, p2, 0.0),
                    axis=0, keepdims=True)
    cntc = jnp.sum(one1.astype(jnp.int32) + one2.astype(jnp.int32),
                   axis=0, keepdims=True)

    @pl.when(g == 0)
    def _():
        loadw_ref[...] = loadc
        cnt_ref[...] = cntc

    @pl.when(g != 0)
    def _():
        loadw_ref[...] += loadc
        cnt_ref[...] += cntc

    @pl.when(g == ng - 1)
    def _():
        load = loadw_ref[...]
        ln = load / jnp.sum(load)
        aux_ref[...] = jnp.sum(ln * jnp.log(ln + 1e-9)).reshape(1, 1)


def _ffn_kernel(se_ref, sb_ref, ss_ref, sen_ref, x_ref, w1_ref, w2_ref,
                o_ref):
    g = pl.program_id(0)
    b = sb_ref[g]
    start = ss_ref[g]
    end = sen_ref[g]
    r = b * BM + jax.lax.broadcasted_iota(jnp.int32, (BM, 1), 0)
    valid = (r >= start) & (r < end)
    xb = x_ref[...]
    h1 = jnp.dot(xb, w1_ref[0], preferred_element_type=jnp.float32)
    a = h1 * jax.nn.sigmoid(h1)
    z = jnp.dot(a, w2_ref[0], preferred_element_type=jnp.float32)
    z = jnp.where(valid, z, 0.0)
    first = jnp.logical_or(g == 0, sb_ref[jnp.maximum(g - 1, 0)] != b)

    @pl.when(first)
    def _():
        o_ref[...] = z

    @pl.when(jnp.logical_not(first))
    def _():
        o_ref[...] += z


def kernel(x, Wg, W1, W2):
    b, t, d = x.shape
    h = x.reshape(-1, d)
    N = h.shape[0]
    E = Wg.shape[1]
    DFF = W1.shape[2]
    P = N * TOPK
    NB = P // BM
    G = NB + E - 1

    ngate = N // BT
    e1, e2, w1, w2, loadw, cnt, aux = pl.pallas_call(
        _gate_kernel,
        grid=(ngate,),
        in_specs=[
            pl.BlockSpec((BT, d), lambda g: (g, 0)),
            pl.BlockSpec((d, E), lambda g: (0, 0)),
        ],
        out_specs=[
            pl.BlockSpec((BT, 1), lambda g: (g, 0)),
            pl.BlockSpec((BT, 1), lambda g: (g, 0)),
            pl.BlockSpec((BT, 1), lambda g: (g, 0)),
            pl.BlockSpec((BT, 1), lambda g: (g, 0)),
            pl.BlockSpec((1, E), lambda g: (0, 0)),
            pl.BlockSpec((1, E), lambda g: (0, 0)),
            pl.BlockSpec((1, 1), lambda g: (0, 0)),
        ],
        out_shape=[
            jax.ShapeDtypeStruct((N, 1), jnp.int32),
            jax.ShapeDtypeStruct((N, 1), jnp.int32),
            jax.ShapeDtypeStruct((N, 1), jnp.float32),
            jax.ShapeDtypeStruct((N, 1), jnp.float32),
            jax.ShapeDtypeStruct((1, E), jnp.float32),
            jax.ShapeDtypeStruct((1, E), jnp.int32),
            jax.ShapeDtypeStruct((1, 1), jnp.float32),
        ],
    )(h, Wg)

    ep = jnp.concatenate([e1, e2], axis=1).reshape(-1)
    wp = jnp.concatenate([w1, w2], axis=1).reshape(-1)
    sort_idx = jnp.argsort(ep)
    tok_sorted = sort_idx // TOPK
    x_sorted = jnp.take(h, tok_sorted, axis=0)

    counts = cnt[0]
    off = jnp.concatenate([jnp.zeros((1,), jnp.int32),
                           jnp.cumsum(counts)]).astype(jnp.int32)

    first_blk = off[:-1] // BM
    last_blk = jnp.maximum(off[1:] - 1, 0) // BM
    nb = jnp.where(counts > 0, last_blk - first_blk + 1, 0)
    cum = jnp.cumsum(nb)
    total = cum[-1]
    gidx = jnp.arange(G, dtype=jnp.int32)
    eg = jnp.searchsorted(cum, gidx, side='right').astype(jnp.int32)
    eg = jnp.minimum(eg, E - 1)
    cum0 = jnp.concatenate([jnp.zeros((1,), jnp.int32),
                            cum.astype(jnp.int32)])
    j = gidx - cum0[eg]
    blk = first_blk[eg] + j
    live = gidx < total
    last_real = jnp.maximum(total - 1, 0)
    se = jnp.where(live, eg, eg[last_real]).astype(jnp.int32)
    sb = jnp.where(live, blk, blk[last_real]).astype(jnp.int32)
    ss = jnp.where(live, jnp.maximum(off[se], sb * BM), 0).astype(jnp.int32)
    sen = jnp.where(live, jnp.minimum(off[se + 1], (sb + 1) * BM),
                    0).astype(jnp.int32)

    out_sorted = pl.pallas_call(
        _ffn_kernel,
        grid_spec=pltpu.PrefetchScalarGridSpec(
            num_scalar_prefetch=4,
            grid=(G,),
            in_specs=[
                pl.BlockSpec((BM, d), lambda g, se, sb, ss, sen: (sb[g], 0)),
                pl.BlockSpec((1, d, DFF),
                             lambda g, se, sb, ss, sen: (se[g], 0, 0)),
                pl.BlockSpec((1, DFF, d),
                             lambda g, se, sb, ss, sen: (se[g], 0, 0)),
            ],
            out_specs=pl.BlockSpec((BM, d),
                                   lambda g, se, sb, ss, sen: (sb[g], 0)),
        ),
        out_shape=jax.ShapeDtypeStruct((P, d), jnp.float32),
        compiler_params=pltpu.CompilerParams(
            dimension_semantics=("arbitrary",)),
    )(se, sb, ss, sen, x_sorted, W1, W2)

    inv = jnp.zeros((P,), jnp.int32).at[sort_idx].set(
        jnp.arange(P, dtype=jnp.int32))
    rows = jnp.take(out_sorted, inv, axis=0).reshape(N, TOPK, d)
    y = jnp.sum(rows * wp.reshape(N, TOPK, 1), axis=1)
    return (y.reshape(b, t, d), aux[0, 0])
